# baseline (speedup 1.0000x reference)
import functools

import jax
import jax.numpy as jnp
from jax import lax
from jax.experimental import pallas as pl
from jax.experimental.pallas import tpu as pltpu
from jax.experimental.pallas import tpu_sc as plsc

EMBED = 32
NC, NS = 2, 16
NW = NC * NS
CHUNK = 128
G = 5
GROWS = G * CHUNK


@functools.lru_cache(maxsize=None)
def _make_kernel(B: int, H: int):
    n_total = B * H
    assert n_total % (NW * CHUNK) == 0
    n_chunks = n_total // (NW * CHUNK)
    assert n_chunks % (2 * G) == 0
    n_groups = n_chunks // G
    nbt = B // CHUNK
    mesh = plsc.VectorSubcoreMesh(core_axis_name="c", subcore_axis_name="s")

    @functools.partial(
        pl.kernel,
        out_type=jax.ShapeDtypeStruct((H, EMBED // 8, nbt, 8, CHUNK),
                                      jnp.float32),
        mesh=mesh,
        scratch_types=[
            pltpu.VMEM((n_chunks, CHUNK), jnp.int32),
            pltpu.VMEM((GROWS, EMBED), jnp.float32),
            pltpu.VMEM((GROWS, EMBED), jnp.float32),
            pltpu.VMEM((G, EMBED // 8, 8, CHUNK), jnp.float32),
            pltpu.VMEM((G, EMBED // 8, 8, CHUNK), jnp.float32),
            pltpu.SemaphoreType.DMA,
            pltpu.SemaphoreType.DMA,
            pltpu.SemaphoreType.DMA,
            pltpu.SemaphoreType.DMA,
        ],
        compiler_params=pltpu.CompilerParams(use_tc_tiling_on_sc=False,
                                             needs_layout_passes=False),
    )
    def body(idx_hbm, table_hbm, out_hbm, idx_v, raw0, raw1, bufT0, bufT1,
             gsem0, gsem1, ssem0, ssem1):
        wid = lax.axis_index("s") * NC + lax.axis_index("c")
        chunk0 = wid * n_chunks
        pltpu.sync_copy(idx_hbm.at[pl.ds(chunk0, n_chunks)], idx_v)
        lanes = lax.broadcasted_iota(jnp.int32, (16,), 0)

        def fire_gathers(grp, raw, sem):
            for c in range(G):
                pltpu.async_copy(table_hbm.at[idx_v.at[grp * G + c]],
                                 raw.at[pl.ds(c * CHUNK, CHUNK)], sem)

        def drain_gathers(raw, sem):
            pltpu.make_async_copy(table_hbm.at[pl.ds(0, GROWS)], raw, sem
                                  ).wait()

        def transpose_group(raw, bufT):
            @pl.loop(0, G * EMBED)
            def _(i):
                c = i // EMBED
                co = i % EMBED
                cols = jnp.broadcast_to(co, (16,))
                for b0 in range(CHUNK // 16):
                    rows = c * CHUNK + b0 * 16 + lanes
                    vals = plsc.load_gather(raw, [rows, cols])
                    bufT[c, co // 8, co % 8, pl.ds(b0 * 16, 16)] = vals

        def store_group(grp, bufT, sem):
            for c in range(G):
                q = (chunk0 + grp * G) + c
                h = q // nbt
                tj = q % nbt
                pltpu.async_copy(bufT.at[c], out_hbm.at[h, :, tj], sem)

        def drain_stores(bufT, sem):
            for c in range(G):
                pltpu.make_async_copy(
                    out_hbm.at[0, :, 0], bufT.at[c], sem).wait()

        def stage(g, raw, bufT, gsem, ssem, first, last):
            drain_gathers(raw, gsem)
            if not first:
                drain_stores(bufT, ssem)
            transpose_group(raw, bufT)
            store_group(g, bufT, ssem)
            if not last:
                fire_gathers(g + 2, raw, gsem)

        fire_gathers(0, raw0, gsem0)
        fire_gathers(1, raw1, gsem1)
        stage(0, raw0, bufT0, gsem0, ssem0, True, False)
        stage(1, raw1, bufT1, gsem1, ssem1, True, False)

        @pl.loop(2, n_groups - 2, step=2)
        def _(g):
            stage(g, raw0, bufT0, gsem0, ssem0, False, False)
            stage(g + 1, raw1, bufT1, gsem1, ssem1, False, False)

        stage(n_groups - 2, raw0, bufT0, gsem0, ssem0, False, True)
        stage(n_groups - 1, raw1, bufT1, gsem1, ssem1, False, True)
        drain_stores(bufT0, ssem0)
        drain_stores(bufT1, ssem1)

    return body


def kernel(x, table):
    B, H = x.shape
    xt = x.T.astype(jnp.int32)
    idx2d = xt.reshape((B * H) // CHUNK, CHUNK)
    out5 = _make_kernel(B, H)(idx2d, table)
    return out5.transpose(2, 4, 0, 1, 3).reshape(B, H, EMBED)


# speedup vs baseline: 1.3220x; 1.3220x over previous
import functools

import jax
import jax.numpy as jnp
from jax import lax
from jax.experimental import pallas as pl
from jax.experimental.pallas import tpu as pltpu
from jax.experimental.pallas import tpu_sc as plsc

EMBED = 32
NC, NS = 2, 16
NW = NC * NS
CHUNK = 128
G = 5
GROWS = G * CHUNK


@functools.lru_cache(maxsize=None)
def _make_kernel(B: int, H: int):
    n_total = B * H
    assert n_total % (NW * CHUNK) == 0
    n_chunks = n_total // (NW * CHUNK)
    assert n_chunks % (2 * G) == 0
    n_groups = n_chunks // G
    nbt = B // CHUNK
    mesh = plsc.VectorSubcoreMesh(core_axis_name="c", subcore_axis_name="s")

    @functools.partial(
        pl.kernel,
        out_type=jax.ShapeDtypeStruct((H, EMBED // 8, nbt, 8, CHUNK),
                                      jnp.float32),
        mesh=mesh,
        scratch_types=[
            pltpu.VMEM((n_chunks, CHUNK), jnp.int32),
            pltpu.VMEM((GROWS, EMBED), jnp.float32),
            pltpu.VMEM((GROWS, EMBED), jnp.float32),
            pltpu.VMEM((G, EMBED // 8, 8, CHUNK), jnp.float32),
            pltpu.VMEM((G, EMBED // 8, 8, CHUNK), jnp.float32),
            pltpu.SemaphoreType.DMA,
            pltpu.SemaphoreType.DMA,
            pltpu.SemaphoreType.DMA,
            pltpu.SemaphoreType.DMA,
        ],
        compiler_params=pltpu.CompilerParams(use_tc_tiling_on_sc=False,
                                             needs_layout_passes=False),
    )
    def body(idx_hbm, table_hbm, out_hbm, idx_v, raw0, raw1, bufT0, bufT1,
             gsem0, gsem1, ssem0, ssem1):
        wid = lax.axis_index("s") * NC + lax.axis_index("c")
        chunk0 = wid * n_chunks
        pltpu.sync_copy(idx_hbm.at[pl.ds(chunk0, n_chunks)], idx_v)
        lanes = lax.broadcasted_iota(jnp.int32, (16,), 0)

        def fire_gathers(grp, raw, sem):
            for c in range(G):
                pltpu.async_copy(table_hbm.at[idx_v.at[grp * G + c]],
                                 raw.at[pl.ds(c * CHUNK, CHUNK)], sem)

        def drain_gathers(raw, sem):
            pltpu.make_async_copy(table_hbm.at[pl.ds(0, GROWS)], raw, sem
                                  ).wait()

        rowvecs = [b0 * 16 + lanes for b0 in range(CHUNK // 16)]

        def transpose_group(raw, bufT):
            @plsc.parallel_loop(0, EMBED)
            def _(co):
                ti = co // 8
                ii = co % 8
                cols = jnp.broadcast_to(co, (16,))
                for c in range(G):
                    sub = raw.at[pl.ds(c * CHUNK, CHUNK)]
                    for b0 in range(CHUNK // 16):
                        vals = plsc.load_gather(sub, [rowvecs[b0], cols])
                        bufT[c, ti, ii, pl.ds(b0 * 16, 16)] = vals

        def store_group(grp, bufT, sem):
            for c in range(G):
                q = (chunk0 + grp * G) + c
                h = q // nbt
                tj = q % nbt
                pltpu.async_copy(bufT.at[c], out_hbm.at[h, :, tj], sem)

        def drain_stores(bufT, sem):
            for c in range(G):
                pltpu.make_async_copy(
                    out_hbm.at[0, :, 0], bufT.at[c], sem).wait()

        def stage(g, raw, bufT, gsem, ssem, first, last):
            drain_gathers(raw, gsem)
            if not first:
                drain_stores(bufT, ssem)
            transpose_group(raw, bufT)
            store_group(g, bufT, ssem)
            if not last:
                fire_gathers(g + 2, raw, gsem)

        fire_gathers(0, raw0, gsem0)
        fire_gathers(1, raw1, gsem1)
        stage(0, raw0, bufT0, gsem0, ssem0, True, False)
        stage(1, raw1, bufT1, gsem1, ssem1, True, False)

        @pl.loop(2, n_groups - 2, step=2)
        def _(g):
            stage(g, raw0, bufT0, gsem0, ssem0, False, False)
            stage(g + 1, raw1, bufT1, gsem1, ssem1, False, False)

        stage(n_groups - 2, raw0, bufT0, gsem0, ssem0, False, True)
        stage(n_groups - 1, raw1, bufT1, gsem1, ssem1, False, True)
        drain_stores(bufT0, ssem0)
        drain_stores(bufT1, ssem1)

    return body


def kernel(x, table):
    B, H = x.shape
    xt = x.T.astype(jnp.int32)
    idx2d = xt.reshape((B * H) // CHUNK, CHUNK)
    out5 = _make_kernel(B, H)(idx2d, table)
    return out5.transpose(2, 4, 0, 1, 3).reshape(B, H, EMBED)


# speedup vs baseline: 1.8058x; 1.3660x over previous
import functools

import jax
import jax.numpy as jnp
from jax import lax
from jax.experimental import pallas as pl
from jax.experimental.pallas import tpu as pltpu
from jax.experimental.pallas import tpu_sc as plsc

EMBED = 32
NC, NS = 2, 16
NW = NC * NS
CHUNK = 128
G = 5
GROWS = G * CHUNK


@functools.lru_cache(maxsize=None)
def _make_kernel(B: int, H: int):
    n_total = B * H
    assert n_total % (NW * CHUNK) == 0
    n_chunks = n_total // (NW * CHUNK)
    assert n_chunks % (2 * G) == 0
    n_groups = n_chunks // G
    nbt = B // CHUNK
    mesh = plsc.VectorSubcoreMesh(core_axis_name="c", subcore_axis_name="s")

    @functools.partial(
        pl.kernel,
        out_type=jax.ShapeDtypeStruct((H, EMBED // 8, nbt, 8, CHUNK),
                                      jnp.float32),
        mesh=mesh,
        scratch_types=[
            pltpu.VMEM((n_chunks, CHUNK), jnp.int32),
            pltpu.VMEM((GROWS, EMBED), jnp.float32),
            pltpu.VMEM((GROWS, EMBED), jnp.float32),
            pltpu.VMEM((G, EMBED // 8, 8, CHUNK + 1), jnp.float32),
            pltpu.VMEM((G, EMBED // 8, 8, CHUNK + 1), jnp.float32),
            pltpu.SemaphoreType.DMA,
            pltpu.SemaphoreType.DMA,
            pltpu.SemaphoreType.DMA,
            pltpu.SemaphoreType.DMA,
        ],
        compiler_params=pltpu.CompilerParams(use_tc_tiling_on_sc=False,
                                             needs_layout_passes=False),
    )
    def body(idx_hbm, table_hbm, out_hbm, idx_v, raw0, raw1, bufT0, bufT1,
             gsem0, gsem1, ssem0, ssem1):
        wid = lax.axis_index("s") * NC + lax.axis_index("c")
        chunk0 = wid * n_chunks
        pltpu.sync_copy(idx_hbm.at[pl.ds(chunk0, n_chunks)], idx_v)
        lanes = lax.broadcasted_iota(jnp.int32, (16,), 0)

        def fire_gathers(grp, raw, sem):
            for c in range(G):
                pltpu.async_copy(table_hbm.at[idx_v.at[grp * G + c]],
                                 raw.at[pl.ds(c * CHUNK, CHUNK)], sem)

        def drain_gathers(raw, sem):
            pltpu.make_async_copy(table_hbm.at[pl.ds(0, GROWS)], raw, sem
                                  ).wait()

        ti_lo = lanes >> 3
        ii_lo = lanes & 7
        ti_hi = ti_lo + 2

        def transpose_group(raw, bufT):
            @plsc.parallel_loop(0, G * CHUNK)
            def _(i):
                c = i // CHUNK
                b = i % CHUNK
                cs = jnp.broadcast_to(c, (16,))
                bs = jnp.broadcast_to(b, (16,))
                lo = raw[i, pl.ds(0, 16)]
                hi = raw[i, pl.ds(16, 16)]
                plsc.store_scatter(bufT, [cs, ti_lo, ii_lo, bs], lo)
                plsc.store_scatter(bufT, [cs, ti_hi, ii_lo, bs], hi)

        def store_group(grp, bufT, sem):
            for c in range(G):
                q = (chunk0 + grp * G) + c
                h = q // nbt
                tj = q % nbt
                pltpu.async_copy(bufT.at[c, :, :, pl.ds(0, CHUNK)],
                                 out_hbm.at[h, :, tj], sem)

        def drain_stores(bufT, sem):
            for c in range(G):
                pltpu.make_async_copy(
                    out_hbm.at[0, :, 0],
                    bufT.at[c, :, :, pl.ds(0, CHUNK)], sem).wait()

        def stage(g, raw, bufT, gsem, ssem, first, last):
            drain_gathers(raw, gsem)
            if not first:
                drain_stores(bufT, ssem)
            transpose_group(raw, bufT)
            store_group(g, bufT, ssem)
            if not last:
                fire_gathers(g + 2, raw, gsem)

        fire_gathers(0, raw0, gsem0)
        fire_gathers(1, raw1, gsem1)
        stage(0, raw0, bufT0, gsem0, ssem0, True, False)
        stage(1, raw1, bufT1, gsem1, ssem1, True, False)

        @pl.loop(2, n_groups - 2, step=2)
        def _(g):
            stage(g, raw0, bufT0, gsem0, ssem0, False, False)
            stage(g + 1, raw1, bufT1, gsem1, ssem1, False, False)

        stage(n_groups - 2, raw0, bufT0, gsem0, ssem0, False, True)
        stage(n_groups - 1, raw1, bufT1, gsem1, ssem1, False, True)
        drain_stores(bufT0, ssem0)
        drain_stores(bufT1, ssem1)

    return body


def kernel(x, table):
    B, H = x.shape
    xt = x.T.astype(jnp.int32)
    idx2d = xt.reshape((B * H) // CHUNK, CHUNK)
    out5 = _make_kernel(B, H)(idx2d, table)
    return out5.transpose(2, 4, 0, 1, 3).reshape(B, H, EMBED)


# speedup vs baseline: 4.7065x; 2.6063x over previous
import functools

import jax
import jax.numpy as jnp
from jax import lax
from jax.experimental import pallas as pl
from jax.experimental.pallas import tpu as pltpu
from jax.experimental.pallas import tpu_sc as plsc

EMBED = 32
NC, NS = 2, 16
NW = NC * NS
CHUNK = 128
G = 5
GROWS = G * CHUNK
TQ = 6
BW = TQ * 128


@functools.lru_cache(maxsize=None)
def _make_tkernel(V: int):
    full = V // 128
    tailr = V - full * 128
    assert tailr % 4 == 0
    ntail = tailr * EMBED // 128
    outrows = full * EMBED + ntail
    NWE = NW - 1
    assert full % NWE == 0
    cols = full // NWE
    assert cols % TQ == 0
    nb = cols // TQ
    assert nb % 2 == 0 and nb >= 4
    mesh = plsc.VectorSubcoreMesh(core_axis_name="c", subcore_axis_name="s")

    @functools.partial(
        pl.kernel,
        out_type=jax.ShapeDtypeStruct((outrows, 128), jnp.float32),
        mesh=mesh,
        scratch_types=[
            pltpu.VMEM((EMBED, BW), jnp.float32),
            pltpu.VMEM((EMBED, BW), jnp.float32),
            pltpu.VMEM((TQ * EMBED, 128), jnp.float32),
            pltpu.VMEM((TQ * EMBED, 128), jnp.float32),
            pltpu.SemaphoreType.DMA,
            pltpu.SemaphoreType.DMA,
            pltpu.SemaphoreType.DMA,
            pltpu.SemaphoreType.DMA,
        ],
        compiler_params=pltpu.CompilerParams(use_tc_tiling_on_sc=True,
                                             needs_layout_passes=False),
    )
    def tbody(tT, tail_lin, out, stgA, stgB, midA, midB, gA, gB, sA, sB):
        wid = lax.axis_index("s") * NC + lax.axis_index("c")
        sw = jnp.minimum(wid, NWE - 1) * cols
        lanes = lax.broadcasted_iota(jnp.int32, (16,), 0)

        def fire_read(b, stg, sem):
            bw = jnp.where(b < nb, b, b - nb)
            pltpu.async_copy(tT.at[:, pl.ds((sw + bw * TQ) * 128, BW)],
                             stg, sem)

        def drain_read(stg, sem):
            pltpu.make_async_copy(tT.at[:, pl.ds(0, BW)], stg, sem).wait()

        def transpose_cols(stg, mid, nq):
            @plsc.parallel_loop(0, 128)
            def _(j):
                vv = (j + lanes) & 127
                for q in range(nq):
                    for c0 in (0, 16):
                        rows = c0 + lanes
                        vals = plsc.load_gather(stg, [rows, q * 128 + vv])
                        f = vv * EMBED + c0 + lanes
                        plsc.store_scatter(
                            mid, [q * EMBED + (f >> 7), f & 127], vals)

        def fire_store(b, mid, sem):
            pltpu.async_copy(
                mid, out.at[pl.ds((sw + b * TQ) * EMBED, TQ * EMBED)], sem)

        def drain_store(mid, sem):
            pltpu.make_async_copy(out.at[pl.ds(0, TQ * EMBED)], mid, sem
                                  ).wait()

        fire_read(0, stgA, gA)
        fire_read(1, stgB, gB)

        @pl.loop(0, nb // 2)
        def _(m):
            k = 2 * m
            drain_read(stgA, gA)
            transpose_cols(stgA, midA, TQ)
            fire_store(k, midA, sA)
            fire_read(k + 2, stgA, gA)
            drain_read(stgB, gB)
            transpose_cols(stgB, midB, TQ)
            fire_store(k + 1, midB, sB)
            fire_read(k + 3, stgB, gB)
            drain_store(midA, sA)
            drain_store(midB, sB)

        drain_read(stgA, gA)
        drain_read(stgB, gB)

        if ntail:
            pltpu.sync_copy(tail_lin, midB.at[pl.ds(0, ntail)])
            pltpu.sync_copy(midB.at[pl.ds(0, ntail)],
                            out.at[pl.ds(full * EMBED, ntail)])

    return tbody


@functools.lru_cache(maxsize=None)
def _make_kernel(B: int, H: int):
    n_total = B * H
    assert n_total % (NW * CHUNK) == 0
    n_chunks = n_total // (NW * CHUNK)
    assert n_chunks % (2 * G) == 0
    n_groups = n_chunks // G
    nbt = B // CHUNK
    mesh = plsc.VectorSubcoreMesh(core_axis_name="c", subcore_axis_name="s")

    @functools.partial(
        pl.kernel,
        out_type=jax.ShapeDtypeStruct((H, EMBED // 8, nbt, 8, CHUNK),
                                      jnp.float32),
        mesh=mesh,
        scratch_types=[
            pltpu.VMEM((n_chunks, CHUNK), jnp.int32),
            pltpu.VMEM((GROWS, EMBED), jnp.float32),
            pltpu.VMEM((GROWS, EMBED), jnp.float32),
            pltpu.VMEM((G, EMBED // 8, 8, CHUNK + 1), jnp.float32),
            pltpu.VMEM((G, EMBED // 8, 8, CHUNK + 1), jnp.float32),
            pltpu.SemaphoreType.DMA,
            pltpu.SemaphoreType.DMA,
            pltpu.SemaphoreType.DMA,
            pltpu.SemaphoreType.DMA,
        ],
        compiler_params=pltpu.CompilerParams(use_tc_tiling_on_sc=False,
                                             needs_layout_passes=False),
    )
    def body(idx_hbm, table_hbm, out_hbm, idx_v, raw0, raw1, bufT0, bufT1,
             gsem0, gsem1, ssem0, ssem1):
        wid = lax.axis_index("s") * NC + lax.axis_index("c")
        chunk0 = wid * n_chunks
        pltpu.sync_copy(idx_hbm.at[pl.ds(chunk0, n_chunks)], idx_v)
        lanes = lax.broadcasted_iota(jnp.int32, (16,), 0)

        def fire_gathers(grp, raw, sem):
            for c in range(G):
                pltpu.async_copy(table_hbm.at[idx_v.at[grp * G + c]],
                                 raw.at[pl.ds(c * CHUNK, CHUNK)], sem)

        def drain_gathers(raw, sem):
            pltpu.make_async_copy(table_hbm.at[pl.ds(0, GROWS)], raw, sem
                                  ).wait()

        ti_lo = lanes >> 3
        ii_lo = lanes & 7
        ti_hi = ti_lo + 2

        def transpose_group(raw, bufT):
            @plsc.parallel_loop(0, G * CHUNK)
            def _(i):
                c = i // CHUNK
                b = i % CHUNK
                cs = jnp.broadcast_to(c, (16,))
                bs = jnp.broadcast_to(b, (16,))
                lo = raw[i, pl.ds(0, 16)]
                hi = raw[i, pl.ds(16, 16)]
                plsc.store_scatter(bufT, [cs, ti_lo, ii_lo, bs], lo)
                plsc.store_scatter(bufT, [cs, ti_hi, ii_lo, bs], hi)

        def store_group(grp, bufT, sem):
            for c in range(G):
                q = (chunk0 + grp * G) + c
                h = q // nbt
                tj = q % nbt
                pltpu.async_copy(bufT.at[c, :, :, pl.ds(0, CHUNK)],
                                 out_hbm.at[h, :, tj], sem)

        def drain_stores(bufT, sem):
            for c in range(G):
                pltpu.make_async_copy(
                    out_hbm.at[0, :, 0],
                    bufT.at[c, :, :, pl.ds(0, CHUNK)], sem).wait()

        def stage(g, raw, bufT, gsem, ssem, first, last):
            drain_gathers(raw, gsem)
            if not first:
                drain_stores(bufT, ssem)
            transpose_group(raw, bufT)
            store_group(g, bufT, ssem)
            if not last:
                fire_gathers(g + 2, raw, gsem)

        fire_gathers(0, raw0, gsem0)
        fire_gathers(1, raw1, gsem1)
        stage(0, raw0, bufT0, gsem0, ssem0, True, False)
        stage(1, raw1, bufT1, gsem1, ssem1, True, False)

        @pl.loop(2, n_groups - 2, step=2)
        def _(g):
            stage(g, raw0, bufT0, gsem0, ssem0, False, False)
            stage(g + 1, raw1, bufT1, gsem1, ssem1, False, False)

        stage(n_groups - 2, raw0, bufT0, gsem0, ssem0, False, True)
        stage(n_groups - 1, raw1, bufT1, gsem1, ssem1, False, True)
        drain_stores(bufT0, ssem0)
        drain_stores(bufT1, ssem1)

    return body


def kernel(x, table):
    B, H = x.shape
    V = table.shape[0]
    full = V // 128
    tailr = V - full * 128
    Vp = full * 128 + (tailr * EMBED // 128) * (128 // EMBED)
    xt = x.T.astype(jnp.int32)
    idx2d = xt.reshape((B * H) // CHUNK, CHUNK)
    tail_lin = table[full * 128:].reshape(tailr * EMBED // 128, 128)
    t250 = _make_tkernel(V)(table.T, tail_lin)
    tab_lin = t250.reshape(Vp, EMBED)
    out5 = _make_kernel(B, H)(idx2d, tab_lin)
    return out5.transpose(2, 4, 0, 1, 3).reshape(B, H, EMBED)


# speedup vs baseline: 5.5659x; 1.1826x over previous
import functools

import jax
import jax.numpy as jnp
from jax import lax
from jax.experimental import pallas as pl
from jax.experimental.pallas import tpu as pltpu
from jax.experimental.pallas import tpu_sc as plsc

EMBED = 32
NC, NS = 2, 16
NW = NC * NS
CHUNK = 128
G = 5
GROWS = G * CHUNK
TQ = 6
BW = TQ * 128


@functools.lru_cache(maxsize=None)
def _make_tkernel(V: int):
    full = V // 128
    tailr = V - full * 128
    assert tailr % 4 == 0
    ntail = tailr * EMBED // 128
    outrows = full * EMBED + ntail
    NWE = NW - 1
    assert full % NWE == 0
    cols = full // NWE
    assert cols % TQ == 0
    nb = cols // TQ
    assert nb % 2 == 0 and nb >= 4
    mesh = plsc.VectorSubcoreMesh(core_axis_name="c", subcore_axis_name="s")

    @functools.partial(
        pl.kernel,
        out_type=jax.ShapeDtypeStruct((outrows, 128), jnp.float32),
        mesh=mesh,
        scratch_types=[
            pltpu.VMEM((EMBED, BW), jnp.float32),
            pltpu.VMEM((EMBED, BW), jnp.float32),
            pltpu.VMEM((TQ * EMBED, 128), jnp.float32),
            pltpu.VMEM((TQ * EMBED, 128), jnp.float32),
            pltpu.SemaphoreType.DMA,
            pltpu.SemaphoreType.DMA,
            pltpu.SemaphoreType.DMA,
            pltpu.SemaphoreType.DMA,
        ],
        compiler_params=pltpu.CompilerParams(use_tc_tiling_on_sc=True,
                                             needs_layout_passes=False),
    )
    def tbody(tT, tail_lin, out, stgA, stgB, midA, midB, gA, gB, sA, sB):
        wid = lax.axis_index("s") * NC + lax.axis_index("c")
        sw = jnp.minimum(wid, NWE - 1) * cols
        lanes = lax.broadcasted_iota(jnp.int32, (16,), 0)

        def fire_read(b, stg, sem):
            bw = jnp.where(b < nb, b, b - nb)
            pltpu.async_copy(tT.at[:, pl.ds((sw + bw * TQ) * 128, BW)],
                             stg, sem)

        def drain_read(stg, sem):
            pltpu.make_async_copy(tT.at[:, pl.ds(0, BW)], stg, sem).wait()

        def transpose_cols(stg, mid, nq):
            @plsc.parallel_loop(0, 128)
            def _(j):
                vv = (j + lanes) & 127
                for q in range(nq):
                    for c0 in (0, 16):
                        rows = c0 + lanes
                        vals = plsc.load_gather(stg, [rows, q * 128 + vv])
                        f = vv * EMBED + c0 + lanes
                        plsc.store_scatter(
                            mid, [q * EMBED + (f >> 7), f & 127], vals)

        def fire_store(b, mid, sem):
            pltpu.async_copy(
                mid, out.at[pl.ds((sw + b * TQ) * EMBED, TQ * EMBED)], sem)

        def drain_store(mid, sem):
            pltpu.make_async_copy(out.at[pl.ds(0, TQ * EMBED)], mid, sem
                                  ).wait()

        fire_read(0, stgA, gA)
        fire_read(1, stgB, gB)

        @pl.loop(0, nb // 2)
        def _(m):
            k = 2 * m
            drain_read(stgA, gA)
            transpose_cols(stgA, midA, TQ)
            fire_store(k, midA, sA)
            fire_read(k + 2, stgA, gA)
            drain_read(stgB, gB)
            transpose_cols(stgB, midB, TQ)
            fire_store(k + 1, midB, sB)
            fire_read(k + 3, stgB, gB)
            drain_store(midA, sA)
            drain_store(midB, sB)

        drain_read(stgA, gA)
        drain_read(stgB, gB)

        if ntail:
            pltpu.sync_copy(tail_lin, midB.at[pl.ds(0, ntail)])
            pltpu.sync_copy(midB.at[pl.ds(0, ntail)],
                            out.at[pl.ds(full * EMBED, ntail)])

    return tbody


@functools.lru_cache(maxsize=None)
def _make_kernel(B: int, H: int):
    n_total = B * H
    assert n_total % (NW * CHUNK) == 0
    n_chunks = n_total // (NW * CHUNK)
    assert n_chunks % (2 * G) == 0
    n_groups = n_chunks // G
    nbt = B // CHUNK
    mesh = plsc.VectorSubcoreMesh(core_axis_name="c", subcore_axis_name="s")

    @functools.partial(
        pl.kernel,
        out_type=jax.ShapeDtypeStruct((H, EMBED // 8, nbt, 8, CHUNK),
                                      jnp.float32),
        mesh=mesh,
        scratch_types=[
            pltpu.VMEM((n_chunks, CHUNK), jnp.int32),
            pltpu.VMEM((GROWS, EMBED), jnp.float32),
            pltpu.VMEM((GROWS, EMBED), jnp.float32),
            pltpu.VMEM((G, EMBED // 8, 8, CHUNK + 1), jnp.float32),
            pltpu.VMEM((G, EMBED // 8, 8, CHUNK + 1), jnp.float32),
            pltpu.SemaphoreType.DMA,
            pltpu.SemaphoreType.DMA,
            pltpu.SemaphoreType.DMA,
            pltpu.SemaphoreType.DMA,
        ],
        compiler_params=pltpu.CompilerParams(use_tc_tiling_on_sc=False,
                                             needs_layout_passes=False),
    )
    def body(idx_hbm, table_hbm, out_hbm, idx_v, raw0, raw1, bufT0, bufT1,
             gsem0, gsem1, ssem0, ssem1):
        wid = lax.axis_index("s") * NC + lax.axis_index("c")
        chunk0 = wid * n_chunks
        pltpu.sync_copy(idx_hbm.at[pl.ds(chunk0, n_chunks)], idx_v)
        lanes = lax.broadcasted_iota(jnp.int32, (16,), 0)

        def fire_gathers(grp, raw, sem):
            for c in range(G):
                pltpu.async_copy(table_hbm.at[idx_v.at[grp * G + c]],
                                 raw.at[pl.ds(c * CHUNK, CHUNK)], sem)

        def drain_gathers(raw, sem):
            pltpu.make_async_copy(table_hbm.at[pl.ds(0, GROWS)], raw, sem
                                  ).wait()

        ti_lo = lanes >> 3
        ii_lo = lanes & 7
        ti_hi = ti_lo + 2
        cs_vecs = [jnp.broadcast_to(jnp.int32(c), (16,)) for c in range(G)]

        def transpose_group(raw, bufT):
            @plsc.parallel_loop(0, CHUNK)
            def _(b):
                bs = jnp.broadcast_to(b, (16,))
                for c in range(G):
                    lo = raw[c * CHUNK + b, pl.ds(0, 16)]
                    hi = raw[c * CHUNK + b, pl.ds(16, 16)]
                    plsc.store_scatter(bufT, [cs_vecs[c], ti_lo, ii_lo, bs],
                                       lo)
                    plsc.store_scatter(bufT, [cs_vecs[c], ti_hi, ii_lo, bs],
                                       hi)

        def store_group(grp, bufT, sem):
            for c in range(G):
                q = (chunk0 + grp * G) + c
                h = q // nbt
                tj = q % nbt
                pltpu.async_copy(bufT.at[c, :, :, pl.ds(0, CHUNK)],
                                 out_hbm.at[h, :, tj], sem)

        def drain_stores(bufT, sem):
            for c in range(G):
                pltpu.make_async_copy(
                    out_hbm.at[0, :, 0],
                    bufT.at[c, :, :, pl.ds(0, CHUNK)], sem).wait()

        def stage(g, raw, bufT, gsem, ssem, first, last):
            drain_gathers(raw, gsem)
            if not first:
                drain_stores(bufT, ssem)
            transpose_group(raw, bufT)
            store_group(g, bufT, ssem)
            if not last:
                fire_gathers(g + 2, raw, gsem)

        fire_gathers(0, raw0, gsem0)
        fire_gathers(1, raw1, gsem1)
        stage(0, raw0, bufT0, gsem0, ssem0, True, False)
        stage(1, raw1, bufT1, gsem1, ssem1, True, False)

        @pl.loop(2, n_groups - 2, step=2)
        def _(g):
            stage(g, raw0, bufT0, gsem0, ssem0, False, False)
            stage(g + 1, raw1, bufT1, gsem1, ssem1, False, False)

        stage(n_groups - 2, raw0, bufT0, gsem0, ssem0, False, True)
        stage(n_groups - 1, raw1, bufT1, gsem1, ssem1, False, True)
        drain_stores(bufT0, ssem0)
        drain_stores(bufT1, ssem1)

    return body


def kernel(x, table):
    B, H = x.shape
    V = table.shape[0]
    full = V // 128
    tailr = V - full * 128
    Vp = full * 128 + (tailr * EMBED // 128) * (128 // EMBED)
    xt = x.T.astype(jnp.int32)
    idx2d = xt.reshape((B * H) // CHUNK, CHUNK)
    tail_lin = table[full * 128:].reshape(tailr * EMBED // 128, 128)
    t250 = _make_tkernel(V)(table.T, tail_lin)
    tab_lin = t250.reshape(Vp, EMBED)
    out5 = _make_kernel(B, H)(idx2d, tab_lin)
    return out5.transpose(2, 4, 0, 1, 3).reshape(B, H, EMBED)
